# baseline (speedup 1.0000x reference)
import functools
import math

import jax
import jax.numpy as jnp
import numpy as np
from jax import lax
from jax.experimental import pallas as pl
from jax.experimental.pallas import tpu as pltpu
from jax.experimental.pallas import tpu_sc as plsc

_BLK = 1024
_SUB = 256


def _bias_segments(seq_q, seq_k, num_buckets=32, max_distance=128):
    d = np.arange(-(seq_q - 1), seq_k)
    n = -d
    offs = (n < 0).astype(np.int32) * (num_buckets // 2)
    n = np.abs(n)
    max_exact = num_buckets // 4
    val = max_exact + (
        np.log(np.maximum(n.astype(np.float32), np.float32(1e-20)) / np.float32(max_exact))
        / np.float32(math.log(max_distance / max_exact))
        * np.float32(num_buckets // 2 - max_exact)
    ).astype(np.int32)
    val = np.minimum(val, num_buckets // 2 - 1)
    bucket = offs + np.where(n < max_exact, n, val)
    segs = []
    start = int(d[0])
    cur = int(bucket[0])
    for k in range(1, len(d)):
        if int(bucket[k]) != cur:
            segs.append((start, int(d[k - 1]), cur))
            start = int(d[k])
            cur = int(bucket[k])
    segs.append((start, int(d[-1]), cur))
    return segs


def _bucket_of_d_np(seq_q, seq_k, num_buckets=32, max_distance=128):
    d = np.arange(-(seq_q - 1), seq_k)
    n = -d
    offs = (n < 0).astype(np.int32) * (num_buckets // 2)
    n = np.abs(n)
    max_exact = num_buckets // 4
    val = max_exact + (
        np.log(np.maximum(n.astype(np.float32), np.float32(1e-20)) / np.float32(max_exact))
        / np.float32(math.log(max_distance / max_exact))
        * np.float32(num_buckets // 2 - max_exact)
    ).astype(np.int32)
    val = np.minimum(val, num_buckets // 2 - 1)
    b = offs + np.where(n < max_exact, n, val)
    pad = (-len(b)) % 8
    return np.concatenate([b, np.full(pad, b[-1], np.int32)]).astype(np.int32)


def _sc_kernel(qk_dots, relative_attention_bias):
    batch, heads, seq_q, seq_k = qk_dots.shape
    assert batch == 1 and heads == 16 and seq_q == 2048
    NC, NS, L = 2, 16, 16
    NW = NC * NS
    total_rows = heads * seq_q
    RPW = total_rows // NW
    RB = 8
    NB = RPW // RB
    RLEN = 3072

    segs = _bias_segments(seq_q, seq_k, relative_attention_bias.shape[0])
    qk2d = qk_dots.reshape(total_rows, seq_k)
    tbl_flat = jnp.pad(relative_attention_bias.reshape(-1), (0, 16))

    mesh = plsc.VectorSubcoreMesh(core_axis_name="c", subcore_axis_name="s")

    @functools.partial(
        pl.kernel,
        mesh=mesh,
        out_type=jax.ShapeDtypeStruct((total_rows, seq_k), jnp.float32),
        scratch_types=[
            pltpu.VMEM((32 * 16 + 16,), jnp.float32),
            pltpu.VMEM((RLEN,), jnp.float32),
            pltpu.VMEM((2, RB, seq_k), jnp.float32),
            pltpu.VMEM((2, RB, seq_k), jnp.float32),
            pltpu.SemaphoreType.DMA((2,)),
            pltpu.SemaphoreType.DMA((2,)),
        ],
    )
    def sck(qk_hbm, tbl_hbm, out_hbm, tbl_v, r_v, inb, outb, insem, outsem):
        w = lax.axis_index("s") * NC + lax.axis_index("c")
        h = w // 2
        i0 = (w % 2) * RPW
        row0 = w * RPW

        pltpu.sync_copy(tbl_hbm, tbl_v)
        tvals = [tbl_v[pl.ds(b * heads + h, L)][0] for b in range(relative_attention_bias.shape[0])]
        lanes = lax.iota(jnp.int32, L)
        dmin = -(i0 + RPW - 1)

        def rbuild(t, carry):
            dvec = dmin + t * L + lanes
            acc = jnp.full((L,), tvals[segs[0][2]], jnp.float32)
            for (ds_, _de, b_) in segs[1:]:
                acc = jnp.where(dvec >= ds_, tvals[b_], acc)
            r_v[pl.ds(t * L, L)] = acc
            return carry

        lax.fori_loop(0, RLEN // L, rbuild, 0)

        def in_copy(b, buf):
            return pltpu.make_async_copy(
                qk_hbm.at[pl.ds(row0 + b * RB, RB)], inb.at[buf], insem.at[buf])

        def out_copy(b, buf):
            return pltpu.make_async_copy(
                outb.at[buf], out_hbm.at[pl.ds(row0 + b * RB, RB)], outsem.at[buf])

        def compute(b, buf):
            def row_body(r8, carry):
                base = (RPW - 1) - (b * RB + r8)
                def col(ci, c2):
                    k = ci * L
                    outb[buf, r8, pl.ds(k, L)] = (
                        inb[buf, r8, pl.ds(k, L)] + r_v[pl.ds(base + k, L)])
                    return c2
                lax.fori_loop(0, seq_k // L, col, 0, unroll=8)
                return carry
            lax.fori_loop(0, RB, row_body, 0)

        in_copy(0, 0).start()
        in_copy(1, 1).start()
        in_copy(0, 0).wait()
        compute(0, 0)
        out_copy(0, 0).start()
        in_copy(2, 0).start()
        in_copy(1, 1).wait()
        compute(1, 1)
        out_copy(1, 1).start()
        in_copy(3, 1).start()

        def steady(bb, carry):
            for b2 in range(2):
                b = bb * 2 + b2
                buf = b2
                in_copy(b, buf).wait()
                out_copy(b - 2, buf).wait()
                compute(b, buf)
                out_copy(b, buf).start()
                in_copy(b + 2, buf).start()
            return carry

        lax.fori_loop(1, NB // 2 - 1, steady, 0)

        for b in (NB - 2, NB - 1):
            buf = b % 2
            in_copy(b, buf).wait()
            out_copy(b - 2, buf).wait()
            compute(b, buf)
            out_copy(b, buf).start()
        out_copy(NB - 2, 0).wait()
        out_copy(NB - 1, 1).wait()

    out = sck(qk2d, tbl_flat)
    return out.reshape(batch, heads, seq_q, seq_k)


def kernel(qk_dots, relative_attention_bias):
    return _sc_kernel(qk_dots, relative_attention_bias)
    batch, heads, seq_q, seq_k = qk_dots.shape
    assert batch == 1 and seq_q % _BLK == 0 and seq_k % _SUB == 0
    ti = seq_q // _BLK
    si = seq_q // _SUB
    sj = seq_k // _SUB
    rpb = _BLK // _SUB
    nd = si + sj - 1

    segs = _bias_segments(seq_q, seq_k, relative_attention_bias.shape[0])

    qk = qk_dots.reshape(heads, seq_q, seq_k)
    tbl = relative_attention_bias.T

    def body(tbl_ref, qk_ref, out_ref, slab_ref):
        h = pl.program_id(0)
        i = pl.program_id(1)

        @pl.when(i == 0)
        def _build_slab():
            ai = jax.lax.broadcasted_iota(jnp.int32, (_SUB, _SUB), 0)
            bi = jax.lax.broadcasted_iota(jnp.int32, (_SUB, _SUB), 1)
            dmat = bi - ai
            for t in range(nd):
                off = (t - (si - 1)) * _SUB
                lo = off - (_SUB - 1)
                hi = off + (_SUB - 1)
                tsegs = [s for s in segs if s[1] >= lo and s[0] <= hi]
                acc = jnp.full((_SUB, _SUB), tbl_ref[h, tsegs[0][2]], jnp.float32)
                for (ds_, _de, b_) in tsegs[1:]:
                    acc = jnp.where(dmat >= (ds_ - off), tbl_ref[h, b_], acc)
                slab_ref[t] = acc

        for it in range(rpb):
            rsl = slice(it * _SUB, (it + 1) * _SUB)
            for jt in range(sj):
                csl = slice(jt * _SUB, (jt + 1) * _SUB)
                t_dyn = jt - (i * rpb + it) + (si - 1)
                out_ref[0, rsl, csl] = qk_ref[0, rsl, csl] + slab_ref[t_dyn]

    out = pl.pallas_call(
        body,
        grid=(heads, ti),
        in_specs=[
            pl.BlockSpec(memory_space=pltpu.SMEM),
            pl.BlockSpec((1, _BLK, seq_k), lambda h, i: (h, i, 0)),
        ],
        out_specs=pl.BlockSpec((1, _BLK, seq_k), lambda h, i: (h, i, 0)),
        out_shape=jax.ShapeDtypeStruct((heads, seq_q, seq_k), jnp.float32),
        scratch_shapes=[pltpu.VMEM((nd, _SUB, _SUB), jnp.float32)],
    )(tbl, qk)
    return out.reshape(batch, heads, seq_q, seq_k)
